# baseline (speedup 1.0000x reference)
import functools

import jax
import jax.numpy as jnp
import numpy as np
from jax import lax
from jax.experimental import pallas as pl
from jax.experimental.pallas import tpu as pltpu
from jax.experimental.pallas import tpu_sc as plsc

NUM_LEVELS = 4
RADIUS = 4
WIN = 2 * RADIUS + 1
CPL = WIN * WIN * NUM_LEVELS

HF = 48
HW = HF * HF
NHEADS = 4
CFEAT = 256

LEVEL_W = (48, 24, 12, 6)
LEVEL_STRIDE = (49, 25, 13, 7)
LEVEL_BASE = (0, 2432, 3072, 3328)
ROW_W = 3456

NHALF = 2
HALF = HW // NHALF
NWORKERS = 32
PIX_PER_W = HALF // NWORKERS
CHUNK = 6
NCHUNKS = PIX_PER_W // CHUNK
OUTROW = CHUNK * CPL
OBUF_STRIDE = OUTROW
PAD_W = 32


def _pool_matrices():
  mats = []
  for i in range(1, NUM_LEVELS):
    s = 2 ** i
    w2 = HF // s
    st = LEVEL_STRIDE[i]
    ncol = -(-(w2 * st) // 128) * 128
    st0 = LEVEL_STRIDE[0]
    p = np.zeros((HF * st0, ncol), np.float32)
    for a in range(w2):
      for b in range(w2):
        col = a * st + b
        for da in range(s):
          for db in range(s):
            p[(s * a + da) * st0 + (s * b + db), col] = 1.0 / (s * s)
    mats.append(p)
  return mats


_P1, _P2, _P3 = _pool_matrices()


def _tc_pyramid_body(f1_ref, f2_ref, p1_ref, p2_ref, p3_ref, out_ref):
  dn = (((0,), (0,)), ((), ()))
  corr = lax.dot_general(
      f1_ref[...], f2_ref[...], dn,
      preferred_element_type=jnp.float32,
      precision=lax.Precision.DEFAULT) * (1.0 / np.sqrt(CFEAT))
  pool_src = corr
  out_ref[:, 0:2432] = jnp.concatenate(
      [corr, jnp.zeros((corr.shape[0], 2432 - 2352), jnp.float32)], axis=1)
  out_ref[:, 2432:3072] = jnp.dot(
      pool_src, p1_ref[...], preferred_element_type=jnp.float32,
      precision=lax.Precision.DEFAULT)
  out_ref[:, 3072:3328] = jnp.dot(
      pool_src, p2_ref[...], preferred_element_type=jnp.float32,
      precision=lax.Precision.DEFAULT)
  out_ref[:, 3328:3456] = jnp.dot(
      pool_src, p3_ref[...], preferred_element_type=jnp.float32,
      precision=lax.Precision.DEFAULT)


def _tc_pyramid(f1, f2, h):
  blk = 384
  grid = HALF // blk
  goff = h * grid
  return pl.pallas_call(
      _tc_pyramid_body,
      grid=(grid,),
      in_specs=[
          pl.BlockSpec((CFEAT, blk), lambda i: (0, i + goff)),
          pl.BlockSpec((CFEAT, HF * LEVEL_STRIDE[0]), lambda i: (0, 0)),
          pl.BlockSpec(_P1.shape, lambda i: (0, 0)),
          pl.BlockSpec(_P2.shape, lambda i: (0, 0)),
          pl.BlockSpec(_P3.shape, lambda i: (0, 0)),
      ],
      out_specs=pl.BlockSpec((blk, ROW_W), lambda i: (i, 0)),
      out_shape=jax.ShapeDtypeStruct((HALF, ROW_W), jnp.float32),
  )(f1, f2, jnp.asarray(_P1), jnp.asarray(_P2), jnp.asarray(_P3))


def _sc_sample_body(h, pyr_hbm, coords_hbm, out_hbm,
                    maps_v0, maps_v1, coords_v, outbuf, sem0, sem1):
  wid = lax.axis_index("s") * 2 + lax.axis_index("c")
  pltpu.sync_copy(
      coords_hbm.at[pl.ds((h * HALF + wid * PIX_PER_W) * 2 * NHEADS,
                          PIX_PER_W * 2 * NHEADS)],
      coords_v)
  lane = lax.iota(jnp.int32, 16)
  zi = jnp.zeros((16,), jnp.int32)
  fz = jnp.zeros((16,), jnp.float32)
  lane9 = lane * 9
  st_mask = lane < WIN
  maps_v0[pl.ds(0, 16)] = fz
  maps_v0[pl.ds(16, 16)] = fz
  maps_v1[pl.ds(0, 16)] = fz
  maps_v1[pl.ds(16, 16)] = fz
  bufs = (maps_v0, maps_v1)
  sems = (sem0, sem1)

  def start_dma(cc):
    p0 = wid * PIX_PER_W + cc * CHUNK
    return pltpu.async_copy(
        pyr_hbm.at[pl.ds(p0 * ROW_W, CHUNK * ROW_W)],
        bufs[cc % 2].at[pl.ds(PAD_W, CHUNK * ROW_W)],
        sems[cc % 2])

  def chunk_compute(cc, maps_v):
    def pix_body(pi, _):
      rowbase = PAD_W + pi * ROW_W
      cbase = (cc * CHUNK + pi) * 2 * NHEADS

      def head_body(n, _):
        c0v = plsc.load_gather(coords_v, [zi + (cbase + 2 * n)])
        c1v = plsc.load_gather(coords_v, [zi + (cbase + 2 * n + 1)])
        obase = n * OBUF_STRIDE + pi * CPL

        @plsc.parallel_loop(0, NUM_LEVELS)
        def lvl_body(l):
          w2 = 48 >> l
          stride = w2 + 1
          base = jnp.where(
              l == 0, 0, jnp.where(l == 1, 2432, jnp.where(l == 2, 3072, 3328)))
          invv = plsc.bitcast(zi + ((127 - l) << 23), jnp.float32)
          cxv = c0v * invv
          cyv = c1v * invv
          x0iv = cxv.astype(jnp.int32)
          x0iv = x0iv - (x0iv.astype(jnp.float32) > cxv).astype(jnp.int32)
          y0iv = cyv.astype(jnp.int32)
          y0iv = y0iv - (y0iv.astype(jnp.float32) > cyv).astype(jnp.int32)
          fxv = cxv - x0iv.astype(jnp.float32)
          fyv = cyv - y0iv.astype(jnp.float32)
          x0s = jnp.max(x0iv)
          y0s = jnp.max(y0iv)
          xs = x0s - RADIUS
          xlv = zi + xs + lane
          xm0 = (xlv >= 0) & (xlv <= w2 - 1)
          xm1 = (xlv >= -1) & (xlv <= w2 - 2)
          addr0 = rowbase + base + xs
          ts = []
          for r in range(WIN + 1):
            yr = y0s + (r - RADIUS)
            ok = (yr >= 0) & (yr <= w2 - 1)
            addr = jnp.where(ok, addr0 + yr * stride, 0)
            idx = zi + addr + lane
            v = plsc.load_gather(maps_v, [idx])
            s = plsc.load_gather(maps_v, [idx + 1])
            v = jnp.where(xm0, v, fz)
            s = jnp.where(xm1, s, fz)
            ts.append(v + fxv * (s - v))
          obl = obase + l * (WIN * WIN)
          for k2 in range(WIN):
            o = ts[k2] + fyv * (ts[k2 + 1] - ts[k2])
            plsc.store_scatter(outbuf, [lane9 + (obl + k2)], o, mask=st_mask)

        return 0

      lax.fori_loop(0, NHEADS, head_body, 0)
      return 0

    lax.fori_loop(0, CHUNK, pix_body, 0)
    cg = wid * NCHUNKS + cc
    for n in range(NHEADS):
      pltpu.sync_copy(outbuf.at[pl.ds(n * OBUF_STRIDE, OUTROW)],
                      out_hbm.at[n, cg])

  pending = start_dma(0)
  for cc in range(NCHUNKS):
    pending.wait()
    if cc + 1 < NCHUNKS:
      pending = start_dma(cc + 1)
    chunk_compute(cc, bufs[cc % 2])


def _sc_sample(pyr, coords_prep, h):
  mesh = plsc.VectorSubcoreMesh(core_axis_name="c", subcore_axis_name="s")
  f = functools.partial(
      pl.kernel,
      out_type=jax.ShapeDtypeStruct((NHEADS, NWORKERS * NCHUNKS, OUTROW),
                                    jnp.float32),
      mesh=mesh,
      compiler_params=pltpu.CompilerParams(
          needs_layout_passes=False, use_tc_tiling_on_sc=False),
      scratch_types=[
          pltpu.VMEM((PAD_W + CHUNK * ROW_W,), jnp.float32),
          pltpu.VMEM((PAD_W + CHUNK * ROW_W,), jnp.float32),
          pltpu.VMEM((PIX_PER_W * 2 * NHEADS,), jnp.float32),
          pltpu.VMEM((NHEADS * OBUF_STRIDE,), jnp.float32),
          pltpu.SemaphoreType.DMA,
          pltpu.SemaphoreType.DMA,
      ],
  )(functools.partial(_sc_sample_body, h))
  return f(pyr.reshape(-1), coords_prep.reshape(-1))


def kernel(fmap1, fmap2, coords):
  B, C, Hf, Wf = fmap1.shape
  _, N, _, H, W = coords.shape
  f1 = fmap1.reshape(C, Hf * Wf)
  f2p = jnp.pad(fmap2.reshape(C, Hf, Wf),
                ((0, 0), (0, 0), (0, LEVEL_STRIDE[0] - Wf))
                ).reshape(C, Hf * LEVEL_STRIDE[0])
  coords_prep = jnp.transpose(coords[0], (2, 3, 0, 1)).reshape(H * W, N * 2)
  halves = []
  for h in range(NHALF):
    pyr = _tc_pyramid(f1, f2p, h)
    halves.append(_sc_sample(pyr, coords_prep, h))
  out = jnp.concatenate(
      [o.reshape(NHEADS, 1, NWORKERS * NCHUNKS * OUTROW) for o in halves],
      axis=1)
  return out.reshape(B, N * CPL, H, W)


# speedup vs baseline: 1.0416x; 1.0416x over previous
import functools

import jax
import jax.numpy as jnp
import numpy as np
from jax import lax
from jax.experimental import pallas as pl
from jax.experimental.pallas import tpu as pltpu
from jax.experimental.pallas import tpu_sc as plsc

NUM_LEVELS = 4
RADIUS = 4
WIN = 2 * RADIUS + 1
CPL = WIN * WIN * NUM_LEVELS

HF = 48
HW = HF * HF
NHEADS = 4
CFEAT = 256

LEVEL_W = (48, 24, 12, 6)
LEVEL_STRIDE = (49, 25, 13, 7)
LEVEL_BASE = (0, 2432, 3072, 3328)
ROW_W = 3456

NWORKERS = 32
PIX_PER_W = HW // NWORKERS
CHUNK = 8
NCHUNKS = PIX_PER_W // CHUNK
OUTROW = CHUNK * CPL
OBUF_STRIDE = OUTROW
PAD_W = 32


def _pool_matrices():
  mats = []
  for i in range(1, NUM_LEVELS):
    s = 2 ** i
    w2 = HF // s
    st = LEVEL_STRIDE[i]
    ncol = -(-(w2 * st) // 128) * 128
    st0 = LEVEL_STRIDE[0]
    p = np.zeros((HF * st0, ncol), np.float32)
    for a in range(w2):
      for b in range(w2):
        col = a * st + b
        for da in range(s):
          for db in range(s):
            p[(s * a + da) * st0 + (s * b + db), col] = 1.0 / (s * s)
    mats.append(p)
  return mats


_P1, _P2, _P3 = _pool_matrices()


def _tc_pyramid_body(f1_ref, f2_ref, p1_ref, p2_ref, p3_ref, out_ref):
  dn = (((0,), (0,)), ((), ()))
  corr = lax.dot_general(
      f1_ref[...], f2_ref[...], dn,
      preferred_element_type=jnp.float32,
      precision=lax.Precision.DEFAULT) * (1.0 / np.sqrt(CFEAT))
  pool_src = corr
  out_ref[:, 0:2432] = jnp.concatenate(
      [corr, jnp.zeros((corr.shape[0], 2432 - 2352), jnp.float32)], axis=1)
  out_ref[:, 2432:3072] = jnp.dot(
      pool_src, p1_ref[...], preferred_element_type=jnp.float32,
      precision=lax.Precision.DEFAULT)
  out_ref[:, 3072:3328] = jnp.dot(
      pool_src, p2_ref[...], preferred_element_type=jnp.float32,
      precision=lax.Precision.DEFAULT)
  out_ref[:, 3328:3456] = jnp.dot(
      pool_src, p3_ref[...], preferred_element_type=jnp.float32,
      precision=lax.Precision.DEFAULT)


def _tc_pyramid(f1, f2):
  blk = 256
  grid = HW // blk
  return pl.pallas_call(
      _tc_pyramid_body,
      grid=(grid,),
      in_specs=[
          pl.BlockSpec((CFEAT, blk), lambda i: (0, i)),
          pl.BlockSpec((CFEAT, HF * LEVEL_STRIDE[0]), lambda i: (0, 0)),
          pl.BlockSpec(_P1.shape, lambda i: (0, 0)),
          pl.BlockSpec(_P2.shape, lambda i: (0, 0)),
          pl.BlockSpec(_P3.shape, lambda i: (0, 0)),
      ],
      out_specs=pl.BlockSpec((blk, ROW_W), lambda i: (i, 0)),
      out_shape=jax.ShapeDtypeStruct((HW, ROW_W), jnp.float32),
  )(f1, f2, jnp.asarray(_P1), jnp.asarray(_P2), jnp.asarray(_P3))


def _sc_sample_body(pyr_hbm, coords_hbm, out_hbm,
                    maps_v0, maps_v1, coords_v, outbuf, sem0, sem1):
  wid = lax.axis_index("s") * 2 + lax.axis_index("c")
  pltpu.sync_copy(
      coords_hbm.at[pl.ds(wid * PIX_PER_W * 2 * NHEADS, PIX_PER_W * 2 * NHEADS)],
      coords_v)
  lane = lax.iota(jnp.int32, 16)
  zi = jnp.zeros((16,), jnp.int32)
  fz = jnp.zeros((16,), jnp.float32)
  lane9 = lane * 9
  st_mask = lane < WIN
  maps_v0[pl.ds(0, 16)] = fz
  maps_v0[pl.ds(16, 16)] = fz
  maps_v1[pl.ds(0, 16)] = fz
  maps_v1[pl.ds(16, 16)] = fz
  bufs = (maps_v0, maps_v1)
  sems = (sem0, sem1)

  def start_dma(cc):
    p0 = wid * PIX_PER_W + cc * CHUNK
    return pltpu.async_copy(
        pyr_hbm.at[pl.ds(p0 * ROW_W, CHUNK * ROW_W)],
        bufs[cc % 2].at[pl.ds(PAD_W, CHUNK * ROW_W)],
        sems[cc % 2])

  def chunk_compute(cc, maps_v):
    def pix_body(pi, _):
      rowbase = PAD_W + pi * ROW_W
      cbase = (cc * CHUNK + pi) * 2 * NHEADS

      def head_body(n, _):
        c0v = plsc.load_gather(coords_v, [zi + (cbase + 2 * n)])
        c1v = plsc.load_gather(coords_v, [zi + (cbase + 2 * n + 1)])
        obase = n * OBUF_STRIDE + pi * CPL

        @plsc.parallel_loop(0, NUM_LEVELS)
        def lvl_body(l):
          w2 = 48 >> l
          stride = w2 + 1
          base = jnp.where(
              l == 0, 0, jnp.where(l == 1, 2432, jnp.where(l == 2, 3072, 3328)))
          invv = plsc.bitcast(zi + ((127 - l) << 23), jnp.float32)
          cxv = c0v * invv
          cyv = c1v * invv
          x0iv = cxv.astype(jnp.int32)
          x0iv = x0iv - (x0iv.astype(jnp.float32) > cxv).astype(jnp.int32)
          y0iv = cyv.astype(jnp.int32)
          y0iv = y0iv - (y0iv.astype(jnp.float32) > cyv).astype(jnp.int32)
          fxv = cxv - x0iv.astype(jnp.float32)
          fyv = cyv - y0iv.astype(jnp.float32)
          x0s = jnp.max(x0iv)
          y0s = jnp.max(y0iv)
          xs = x0s - RADIUS
          xlv = zi + xs + lane
          xm0 = (xlv >= 0) & (xlv <= w2 - 1)
          xm1 = (xlv >= -1) & (xlv <= w2 - 2)
          addr0 = rowbase + base + xs
          ts = []
          for r in range(WIN + 1):
            yr = y0s + (r - RADIUS)
            ok = (yr >= 0) & (yr <= w2 - 1)
            addr = jnp.where(ok, addr0 + yr * stride, 0)
            idx = zi + addr + lane
            v = plsc.load_gather(maps_v, [idx])
            s = plsc.load_gather(maps_v, [idx + 1])
            v = jnp.where(xm0, v, fz)
            s = jnp.where(xm1, s, fz)
            ts.append(v + fxv * (s - v))
          obl = obase + l * (WIN * WIN)
          for k2 in range(WIN):
            o = ts[k2] + fyv * (ts[k2 + 1] - ts[k2])
            plsc.store_scatter(outbuf, [lane9 + (obl + k2)], o, mask=st_mask)

        return 0

      lax.fori_loop(0, NHEADS, head_body, 0)
      return 0

    lax.fori_loop(0, CHUNK, pix_body, 0)
    cg = wid * NCHUNKS + cc
    for n in range(NHEADS):
      pltpu.sync_copy(outbuf.at[pl.ds(n * OBUF_STRIDE, OUTROW)],
                      out_hbm.at[n, cg])

  pending = start_dma(0)
  for cc in range(NCHUNKS):
    pending.wait()
    if cc + 1 < NCHUNKS:
      pending = start_dma(cc + 1)
    chunk_compute(cc, bufs[cc % 2])


def _sc_sample(pyr, coords_prep):
  mesh = plsc.VectorSubcoreMesh(core_axis_name="c", subcore_axis_name="s")
  f = functools.partial(
      pl.kernel,
      out_type=jax.ShapeDtypeStruct((NHEADS, NWORKERS * NCHUNKS, OUTROW),
                                    jnp.float32),
      mesh=mesh,
      compiler_params=pltpu.CompilerParams(
          needs_layout_passes=False, use_tc_tiling_on_sc=False),
      scratch_types=[
          pltpu.VMEM((PAD_W + CHUNK * ROW_W,), jnp.float32),
          pltpu.VMEM((PAD_W + CHUNK * ROW_W,), jnp.float32),
          pltpu.VMEM((PIX_PER_W * 2 * NHEADS,), jnp.float32),
          pltpu.VMEM((NHEADS * OBUF_STRIDE,), jnp.float32),
          pltpu.SemaphoreType.DMA,
          pltpu.SemaphoreType.DMA,
      ],
  )(_sc_sample_body)
  return f(pyr.reshape(-1), coords_prep.reshape(-1))


def kernel(fmap1, fmap2, coords):
  B, C, Hf, Wf = fmap1.shape
  _, N, _, H, W = coords.shape
  f1 = fmap1.reshape(C, Hf * Wf)
  f2p = jnp.pad(fmap2.reshape(C, Hf, Wf),
                ((0, 0), (0, 0), (0, LEVEL_STRIDE[0] - Wf))
                ).reshape(C, Hf * LEVEL_STRIDE[0])
  pyr = _tc_pyramid(f1, f2p)
  coords_prep = jnp.transpose(coords[0], (2, 3, 0, 1)).reshape(H * W, N * 2)
  out = _sc_sample(pyr, coords_prep)
  return out.reshape(B, N * CPL, H, W)
